# baseline (speedup 1.0000x reference)
import functools

import jax
import jax.numpy as jnp
from jax import lax
from jax.experimental import pallas as pl
from jax.experimental.pallas import tpu as pltpu
from jax.experimental.pallas import tpu_sc as plsc

L = 16
BCHUNK = 4096
UNROLL = 16


def kernel(dom_idx, paras):
  B = dom_idx.shape[0]
  V, D = paras.shape
  table_t = paras.T
  info = plsc.get_sparse_core_info()
  nc = info.num_cores
  nw = nc * info.num_subcores
  c_per_w = D // nw
  n_chunks = B // BCHUNK
  n_groups = BCHUNK // L

  mesh = plsc.VectorSubcoreMesh(core_axis_name="c", subcore_axis_name="s")

  @functools.partial(
      pl.kernel,
      mesh=mesh,
      out_type=jax.ShapeDtypeStruct((D, B), jnp.float32),
      compiler_params=pltpu.CompilerParams(needs_layout_passes=False),
      scratch_types=[
          pltpu.VMEM((B,), jnp.int32),
          pltpu.VMEM((1, V), jnp.float32),
          pltpu.VMEM((2, BCHUNK), jnp.float32),
          pltpu.SemaphoreType.DMA,
          pltpu.SemaphoreType.DMA,
      ],
  )
  def gather_kernel(idx_hbm, table_hbm, out_hbm, idx_v, chan_v, outbuf_v,
                    chan_sem, out_sem):
    wid = lax.axis_index("s") * nc + lax.axis_index("c")
    zeros = jnp.zeros((L,), jnp.int32)

    def chan_copy(j):
      return pltpu.make_async_copy(
          table_hbm.at[pl.ds(wid * c_per_w + j, 1), :], chan_v, chan_sem)

    chan_copy(0).start()
    pltpu.sync_copy(idx_hbm.at[...], idx_v)

    for j in range(c_per_w):
      ch = wid * c_per_w + j
      chan_copy(j).wait()
      for k in range(n_chunks):
        buf = (j * n_chunks + k) % 2

        def gather_group(g16, carry, k=k, buf=buf):
          for u in range(UNROLL):
            g = g16 * UNROLL + u
            v = idx_v[pl.ds(k * BCHUNK + g * L, L)]
            outbuf_v[buf, pl.ds(g * L, L)] = plsc.load_gather(
                chan_v, [zeros, v])
          return carry

        if j * n_chunks + k >= 2:
          pltpu.make_async_copy(
              outbuf_v.at[pl.ds(buf, 1)],
              out_hbm.at[pl.ds(0, 1), pl.ds(0, BCHUNK)],
              out_sem,
          ).wait()
        lax.fori_loop(0, n_groups // UNROLL, gather_group, 0)
        pltpu.async_copy(
            outbuf_v.at[pl.ds(buf, 1)],
            out_hbm.at[pl.ds(ch, 1), pl.ds(k * BCHUNK, BCHUNK)],
            out_sem,
        )
        if j == 0 and k == n_chunks - 1:
          chan_copy(1).start()
    for _ in range(2):
      pltpu.make_async_copy(
          outbuf_v.at[pl.ds(0, 1)],
          out_hbm.at[pl.ds(0, 1), pl.ds(0, BCHUNK)],
          out_sem,
      ).wait()

  out_t = gather_kernel(dom_idx, table_t)
  return out_t.T


# speedup vs baseline: 1.0054x; 1.0054x over previous
import functools

import jax
import jax.numpy as jnp
from jax import lax
from jax.experimental import pallas as pl
from jax.experimental.pallas import tpu as pltpu
from jax.experimental.pallas import tpu_sc as plsc

L = 16
BCHUNK = 4096
UNROLL = 8


def kernel(dom_idx, paras):
  B = dom_idx.shape[0]
  V, D = paras.shape
  table_t = paras.T
  info = plsc.get_sparse_core_info()
  nc = info.num_cores
  nw = nc * info.num_subcores
  c_per_w = D // nw
  n_chunks = B // BCHUNK
  n_groups = BCHUNK // L

  mesh = plsc.VectorSubcoreMesh(core_axis_name="c", subcore_axis_name="s")

  @functools.partial(
      pl.kernel,
      mesh=mesh,
      out_type=jax.ShapeDtypeStruct((D, B), jnp.float32),
      compiler_params=pltpu.CompilerParams(needs_layout_passes=False),
      scratch_types=[
          pltpu.VMEM((B,), jnp.int32),
          pltpu.VMEM((1, V), jnp.float32),
          pltpu.VMEM((2, BCHUNK), jnp.float32),
          pltpu.SemaphoreType.DMA,
          pltpu.SemaphoreType.DMA,
      ],
  )
  def gather_kernel(idx_hbm, table_hbm, out_hbm, idx_v, chan_v, outbuf_v,
                    chan_sem, out_sem):
    wid = lax.axis_index("s") * nc + lax.axis_index("c")
    zeros = jnp.zeros((L,), jnp.int32)

    def chan_copy(j):
      return pltpu.make_async_copy(
          table_hbm.at[pl.ds(wid * c_per_w + j, 1), :], chan_v, chan_sem)

    chan_copy(0).start()
    pltpu.sync_copy(idx_hbm.at[...], idx_v)

    for j in range(c_per_w):
      ch = wid * c_per_w + j
      chan_copy(j).wait()
      for k in range(n_chunks):
        buf = (j * n_chunks + k) % 2

        def gather_group(g16, carry, k=k, buf=buf):
          for u in range(UNROLL):
            g = g16 * UNROLL + u
            v = idx_v[pl.ds(k * BCHUNK + g * L, L)]
            outbuf_v[buf, pl.ds(g * L, L)] = plsc.load_gather(
                chan_v, [zeros, v])
          return carry

        if j * n_chunks + k >= 2:
          pltpu.make_async_copy(
              outbuf_v.at[pl.ds(buf, 1)],
              out_hbm.at[pl.ds(0, 1), pl.ds(0, BCHUNK)],
              out_sem,
          ).wait()
        lax.fori_loop(0, n_groups // UNROLL, gather_group, 0)
        pltpu.async_copy(
            outbuf_v.at[pl.ds(buf, 1)],
            out_hbm.at[pl.ds(ch, 1), pl.ds(k * BCHUNK, BCHUNK)],
            out_sem,
        )
        if j == 0 and k == n_chunks - 1:
          chan_copy(1).start()
    for _ in range(2):
      pltpu.make_async_copy(
          outbuf_v.at[pl.ds(0, 1)],
          out_hbm.at[pl.ds(0, 1), pl.ds(0, BCHUNK)],
          out_sem,
      ).wait()

  out_t = gather_kernel(dom_idx, table_t)
  return out_t.T


# speedup vs baseline: 1.1682x; 1.1619x over previous
import functools

import jax
import jax.numpy as jnp
from jax import lax
from jax.experimental import pallas as pl
from jax.experimental.pallas import tpu as pltpu
from jax.experimental.pallas import tpu_sc as plsc

L = 16
BCHUNK = 4096
UNROLL = 16


def kernel(dom_idx, paras):
  B = dom_idx.shape[0]
  V, D = paras.shape
  table_t = paras.T
  info = plsc.get_sparse_core_info()
  nc = info.num_cores
  nw = nc * info.num_subcores
  c_per_w = D // nw
  n_chunks = B // BCHUNK
  n_groups = BCHUNK // L

  mesh = plsc.VectorSubcoreMesh(core_axis_name="c", subcore_axis_name="s")

  @functools.partial(
      pl.kernel,
      mesh=mesh,
      out_type=jax.ShapeDtypeStruct((D, B), jnp.float32),
      compiler_params=pltpu.CompilerParams(needs_layout_passes=False),
      scratch_types=[
          pltpu.VMEM((B,), jnp.int32),
          pltpu.VMEM((1, V), jnp.float32),
          pltpu.VMEM((1, BCHUNK), jnp.float32),
          pltpu.SemaphoreType.DMA,
      ],
  )
  def gather_kernel(idx_hbm, table_hbm, out_hbm, idx_v, chan_v, outbuf_v,
                    sem):
    wid = lax.axis_index("s") * nc + lax.axis_index("c")
    pltpu.sync_copy(idx_hbm.at[...], idx_v)
    zeros = jnp.zeros((L,), jnp.int32)

    for j in range(c_per_w):
      ch = wid * c_per_w + j
      pltpu.sync_copy(table_hbm.at[pl.ds(ch, 1), :], chan_v)
      for k in range(n_chunks):

        def gather_group(gu, carry, k=k):
          for u in range(UNROLL):
            g = gu * UNROLL + u
            v = idx_v[pl.ds(k * BCHUNK + g * L, L)]
            outbuf_v[0, pl.ds(g * L, L)] = plsc.load_gather(
                chan_v, [zeros, v])
          return carry

        lax.fori_loop(0, n_groups // UNROLL, gather_group, 0)
        pltpu.sync_copy(
            outbuf_v, out_hbm.at[pl.ds(ch, 1), pl.ds(k * BCHUNK, BCHUNK)]
        )

  out_t = gather_kernel(dom_idx, table_t)
  return out_t.T


# speedup vs baseline: 1.2051x; 1.0316x over previous
import functools

import jax
import jax.numpy as jnp
from jax import lax
from jax.experimental import pallas as pl
from jax.experimental.pallas import tpu as pltpu
from jax.experimental.pallas import tpu_sc as plsc

L = 16
BCHUNK = 8192
UNROLL = 8


def kernel(dom_idx, paras):
  B = dom_idx.shape[0]
  V, D = paras.shape
  table_t = paras.T
  info = plsc.get_sparse_core_info()
  nc = info.num_cores
  nw = nc * info.num_subcores
  c_per_w = D // nw
  n_chunks = B // BCHUNK
  n_groups = BCHUNK // L

  mesh = plsc.VectorSubcoreMesh(core_axis_name="c", subcore_axis_name="s")

  @functools.partial(
      pl.kernel,
      mesh=mesh,
      out_type=jax.ShapeDtypeStruct((D, B), jnp.float32),
      compiler_params=pltpu.CompilerParams(needs_layout_passes=False),
      scratch_types=[
          pltpu.VMEM((B,), jnp.int32),
          pltpu.VMEM((1, V), jnp.float32),
          pltpu.VMEM((1, BCHUNK), jnp.float32),
          pltpu.SemaphoreType.DMA,
      ],
  )
  def gather_kernel(idx_hbm, table_hbm, out_hbm, idx_v, chan_v, outbuf_v,
                    sem):
    wid = lax.axis_index("s") * nc + lax.axis_index("c")
    pltpu.sync_copy(idx_hbm.at[...], idx_v)
    zeros = jnp.zeros((L,), jnp.int32)

    for j in range(c_per_w):
      ch = wid * c_per_w + j
      pltpu.sync_copy(table_hbm.at[pl.ds(ch, 1), :], chan_v)
      for k in range(n_chunks):

        def gather_group(gu, carry, k=k):
          for u in range(UNROLL):
            g = gu * UNROLL + u
            v = idx_v[pl.ds(k * BCHUNK + g * L, L)]
            outbuf_v[0, pl.ds(g * L, L)] = plsc.load_gather(
                chan_v, [zeros, v])
          return carry

        lax.fori_loop(0, n_groups // UNROLL, gather_group, 0)
        pltpu.sync_copy(
            outbuf_v, out_hbm.at[pl.ds(ch, 1), pl.ds(k * BCHUNK, BCHUNK)]
        )

  out_t = gather_kernel(dom_idx, table_t)
  return out_t.T


# speedup vs baseline: 1.2099x; 1.0039x over previous
import functools

import jax
import jax.numpy as jnp
from jax import lax
from jax.experimental import pallas as pl
from jax.experimental.pallas import tpu as pltpu
from jax.experimental.pallas import tpu_sc as plsc

L = 16
BCHUNK = 8192
UNROLL = 8


def kernel(dom_idx, paras):
  B = dom_idx.shape[0]
  V, D = paras.shape
  table_t = paras.T
  info = plsc.get_sparse_core_info()
  nc = info.num_cores
  nw = nc * info.num_subcores
  c_per_w = D // nw
  n_chunks = B // BCHUNK
  n_groups = BCHUNK // L

  mesh = plsc.VectorSubcoreMesh(core_axis_name="c", subcore_axis_name="s")

  @functools.partial(
      pl.kernel,
      mesh=mesh,
      out_type=jax.ShapeDtypeStruct((D, B), jnp.float32),
      compiler_params=pltpu.CompilerParams(
          needs_layout_passes=False,
          disable_bounds_checks=True,
          disable_semaphore_checks=True,
      ),
      scratch_types=[
          pltpu.VMEM((B,), jnp.int32),
          pltpu.VMEM((1, V), jnp.float32),
          pltpu.VMEM((1, BCHUNK), jnp.float32),
          pltpu.SemaphoreType.DMA,
      ],
  )
  def gather_kernel(idx_hbm, table_hbm, out_hbm, idx_v, chan_v, outbuf_v,
                    sem):
    wid = lax.axis_index("s") * nc + lax.axis_index("c")
    pltpu.sync_copy(idx_hbm.at[...], idx_v)
    zeros = jnp.zeros((L,), jnp.int32)

    for j in range(c_per_w):
      ch = wid * c_per_w + j
      pltpu.sync_copy(table_hbm.at[pl.ds(ch, 1), :], chan_v)
      for k in range(n_chunks):

        def gather_group(gu, carry, k=k):
          for u in range(UNROLL):
            g = gu * UNROLL + u
            v = idx_v[pl.ds(k * BCHUNK + g * L, L)]
            outbuf_v[0, pl.ds(g * L, L)] = plsc.load_gather(
                chan_v, [zeros, v])
          return carry

        lax.fori_loop(0, n_groups // UNROLL, gather_group, 0)
        pltpu.sync_copy(
            outbuf_v, out_hbm.at[pl.ds(ch, 1), pl.ds(k * BCHUNK, BCHUNK)]
        )

  out_t = gather_kernel(dom_idx, table_t)
  return out_t.T
